# baseline (speedup 1.0000x reference)
import functools

import jax
import jax.numpy as jnp
from jax import lax
from jax.experimental import pallas as pl
from jax.experimental.pallas import tpu as pltpu
from jax.experimental.pallas import tpu_sc as plsc

_NUM_FIELDS = 26
_VOCAB = 100000
_EMB = 64
_BATCH = 4096

_NC = 2
_NS = 16
_NW = _NC * _NS
_ROWS = _BATCH * _NUM_FIELDS
_RPW = _ROWS // _NW
_CHUNK = 128
_NCHUNK = _RPW // _CHUNK
_LANES = 16


def _body(table, xh, offh, out, xv, offv, idxv, rows, sem):
    wid = lax.axis_index("s") * _NC + lax.axis_index("c")
    base = wid * _RPW

    pltpu.sync_copy(xh.at[pl.ds(base, _RPW)], xv)
    pltpu.sync_copy(offh, offv)

    def add_body(i, carry):
        sl = pl.ds(i * _LANES, _LANES)
        idxv[sl] = xv[sl] + offv[sl]
        return carry

    lax.fori_loop(0, _RPW // _LANES, add_body, 0)

    def gather_body(c, carry):
        idx_sl = idxv.at[pl.ds(c * _CHUNK, _CHUNK)]
        pltpu.async_copy(table.at[idx_sl], rows, sem).wait()
        pltpu.sync_copy(rows, out.at[pl.ds(base + c * _CHUNK, _CHUNK)])
        return carry

    lax.fori_loop(0, _NCHUNK, gather_body, 0)


@jax.jit
def _sc_gather(table_flat, x_flat, offs):
    mesh = plsc.VectorSubcoreMesh(core_axis_name="c", subcore_axis_name="s")
    f = functools.partial(
        pl.kernel,
        mesh=mesh,
        out_type=jax.ShapeDtypeStruct((_ROWS, _EMB), jnp.float32),
        scratch_types=[
            pltpu.VMEM((_RPW,), jnp.int32),
            pltpu.VMEM((_RPW,), jnp.int32),
            pltpu.VMEM((_RPW,), jnp.int32),
            pltpu.VMEM((_CHUNK, _EMB), jnp.float32),
            pltpu.SemaphoreType.DMA,
        ],
        compiler_params=pltpu.CompilerParams(use_tc_tiling_on_sc=False),
    )(_body)
    return f(table_flat, x_flat, offs)


def kernel(X, tables):
    table_flat = tables.reshape(_NUM_FIELDS * _VOCAB, _EMB)
    x_flat = X.reshape(-1)
    offs = jnp.tile(
        jnp.arange(_NUM_FIELDS, dtype=jnp.int32) * _VOCAB, _RPW // _NUM_FIELDS
    )
    out = _sc_gather(table_flat, x_flat, offs)
    return out.reshape(_BATCH, 1, _NUM_FIELDS * _EMB)


# speedup vs baseline: 1.0121x; 1.0121x over previous
import functools

import jax
import jax.numpy as jnp
from jax import lax
from jax.experimental import pallas as pl
from jax.experimental.pallas import tpu as pltpu
from jax.experimental.pallas import tpu_sc as plsc

_NUM_FIELDS = 26
_VOCAB = 100000
_EMB = 64
_BATCH = 4096

_NC = 2
_NS = 16
_NW = _NC * _NS
_ROWS = _BATCH * _NUM_FIELDS
_RPW = _ROWS // _NW
_CHUNK = 104
_NCHUNK = _RPW // _CHUNK
_NBUF = 8
_NROUND = _NCHUNK // _NBUF
_LANES = 16


def _body(table, xh, offh, out, xv, offv, rows, *sems):
    wid = lax.axis_index("s") * _NC + lax.axis_index("c")
    base = wid * _RPW

    pltpu.sync_copy(xh.at[pl.ds(base, _RPW)], xv)
    pltpu.sync_copy(offh, offv)

    def add_body(i, carry):
        sl = pl.ds(i * _LANES, _LANES)
        xv[sl] = xv[sl] + offv[sl]
        return carry

    lax.fori_loop(0, _RPW // _LANES, add_body, 0)

    def chunk_off(g):
        return pl.multiple_of(g * _CHUNK, 8)

    def fire_gather(g, b):
        pltpu.async_copy(
            table.at[xv.at[pl.ds(chunk_off(g), _CHUNK)]], rows.at[b], sems[b]
        )

    def wait_gather(g, b):
        pltpu.make_async_copy(
            table.at[xv.at[pl.ds(chunk_off(g), _CHUNK)]], rows.at[b], sems[b]
        ).wait()

    def fire_wb(g, b):
        pltpu.async_copy(
            rows.at[b], out.at[pl.ds(base + chunk_off(g), _CHUNK)], sems[b]
        )

    def wait_wb(g, b):
        pltpu.make_async_copy(
            rows.at[b], out.at[pl.ds(base + chunk_off(g), _CHUNK)], sems[b]
        ).wait()

    for b in range(_NBUF):
        fire_gather(b, b)

    def round_body(t, carry):
        for b in range(_NBUF):
            g = t * _NBUF + b
            wait_gather(g, b)
            fire_wb(g, b)
            wait_wb(g, b)
            fire_gather(g + _NBUF, b)
        return carry

    lax.fori_loop(0, _NROUND - 1, round_body, 0)

    for b in range(_NBUF):
        g = (_NROUND - 1) * _NBUF + b
        wait_gather(g, b)
        fire_wb(g, b)
        wait_wb(g, b)


@jax.jit
def _sc_gather(table_flat, x_flat, offs):
    mesh = plsc.VectorSubcoreMesh(core_axis_name="c", subcore_axis_name="s")
    f = functools.partial(
        pl.kernel,
        mesh=mesh,
        out_type=jax.ShapeDtypeStruct((_ROWS, _EMB), jnp.float32),
        scratch_types=[
            pltpu.VMEM((_RPW,), jnp.int32),
            pltpu.VMEM((_RPW,), jnp.int32),
            pltpu.VMEM((_NBUF, _CHUNK, _EMB), jnp.float32),
        ]
        + [pltpu.SemaphoreType.DMA] * _NBUF,
        compiler_params=pltpu.CompilerParams(use_tc_tiling_on_sc=False),
    )(_body)
    return f(table_flat, x_flat, offs)


def kernel(X, tables):
    table_flat = tables.reshape(_NUM_FIELDS * _VOCAB, _EMB)
    x_flat = X.reshape(-1)
    offs = jnp.tile(
        jnp.arange(_NUM_FIELDS, dtype=jnp.int32) * _VOCAB, _RPW // _NUM_FIELDS
    )
    out = _sc_gather(table_flat, x_flat, offs)
    return out.reshape(_BATCH, 1, _NUM_FIELDS * _EMB)
